# baseline (device time: 116759 ns/iter reference)
import jax
import jax.numpy as jnp
from jax import lax
from jax.experimental import pallas as pl
from jax.experimental.pallas import tpu as pltpu

N_DEV = 16
N_TOK = 1024
D_IN = 512
D_OUT = 1024
E_LOCAL = 4
CAP = 12
CHUNK = N_TOK // N_DEV
N_STEP = N_DEV - 1


def _body(x_ref, keep_ref, w_ref, out_ref,
          accb_ref, outb_ref, rsrecv_ref, rssend_ref,
          rs_ssem, rs_rsem, ag_ssem, ag_rsem):
    my = lax.axis_index("i")
    left = lax.rem(my + N_DEV - 1, N_DEV)
    right = lax.rem(my + 1, N_DEV)

    barrier = pltpu.get_barrier_semaphore()
    for nbr in (left, right):
        pl.semaphore_signal(barrier, inc=1, device_id=(nbr,),
                            device_id_type=pl.DeviceIdType.MESH)
    pl.semaphore_wait(barrier, 2)

    x = x_ref[...]
    acc = jnp.zeros((N_TOK, D_OUT), jnp.float32)
    for k in range(E_LOCAL):
        xk = x * keep_ref[:, k:k + 1]
        acc += jnp.dot(xk, w_ref[k], preferred_element_type=jnp.float32)
    accb_ref[...] = acc.astype(jnp.bfloat16)

    def cslice(idx):
        return pl.ds(lax.rem(idx + 2 * N_DEV, N_DEV) * CHUNK, CHUNK)

    for h in range(N_STEP):
        if h == 0:
            rssend_ref[0] = accb_ref[cslice(my), :]
        else:
            rssend_ref[h] = rsrecv_ref[h - 1] + accb_ref[cslice(my - h), :]
        rdma = pltpu.make_async_remote_copy(
            src_ref=rssend_ref.at[h],
            dst_ref=rsrecv_ref.at[h],
            send_sem=rs_ssem.at[h],
            recv_sem=rs_rsem.at[h],
            device_id=(right,),
            device_id_type=pl.DeviceIdType.MESH,
        )
        rdma.start()
        rdma.wait()

    outb_ref[cslice(my + 1), :] = (
        rsrecv_ref[N_STEP - 1] + accb_ref[cslice(my + 1), :]
    )

    for g in range(N_STEP):
        rdma = pltpu.make_async_remote_copy(
            src_ref=outb_ref.at[cslice(my + 1 - g)],
            dst_ref=outb_ref.at[cslice(my + 1 - g)],
            send_sem=ag_ssem.at[g],
            recv_sem=ag_rsem.at[g],
            device_id=(right,),
            device_id_type=pl.DeviceIdType.MESH,
        )
        rdma.start()
        rdma.wait()

    out_ref[...] = outb_ref[...].astype(jnp.float32)


def kernel(x, router_W, route_idx, expert_W):
    del router_W
    my = lax.axis_index("i")
    route = route_idx[:, 0]
    eids = my * E_LOCAL + jnp.arange(E_LOCAL, dtype=route.dtype)
    sel = route[:, None] == eids[None, :]
    occ = jnp.cumsum(sel.astype(jnp.int32), axis=0)
    keep = (sel & (occ <= CAP)).astype(jnp.bfloat16)
    xb = x.astype(jnp.bfloat16)
    wb = expert_W.astype(jnp.bfloat16)

    return pl.pallas_call(
        _body,
        out_shape=jax.ShapeDtypeStruct((N_TOK, D_OUT), jnp.float32),
        in_specs=[pl.BlockSpec(memory_space=pltpu.VMEM)] * 3,
        out_specs=pl.BlockSpec(memory_space=pltpu.VMEM),
        scratch_shapes=[
            pltpu.VMEM((N_TOK, D_OUT), jnp.bfloat16),
            pltpu.VMEM((N_TOK, D_OUT), jnp.bfloat16),
            pltpu.VMEM((N_STEP, CHUNK, D_OUT), jnp.bfloat16),
            pltpu.VMEM((N_STEP, CHUNK, D_OUT), jnp.bfloat16),
            pltpu.SemaphoreType.DMA((N_STEP,)),
            pltpu.SemaphoreType.DMA((N_STEP,)),
            pltpu.SemaphoreType.DMA((N_STEP,)),
            pltpu.SemaphoreType.DMA((N_STEP,)),
        ],
        compiler_params=pltpu.CompilerParams(collective_id=0),
    )(xb, keep, wb)


# device time: 61012 ns/iter; 1.9137x vs baseline; 1.9137x over previous
import jax
import jax.numpy as jnp
from jax import lax
from jax.experimental import pallas as pl
from jax.experimental.pallas import tpu as pltpu

N_DEV = 16
N_TOK = 1024
D_IN = 512
D_OUT = 1024
N_EXP = 64
E_LOCAL = 4
CAP = 12
SLOCAL = E_LOCAL * CAP
STOT = N_DEV * SLOCAL


def _body(xgm_ref, w_ref, p_ref, out_ref, ag_ref, ssem, rsem):
    my = lax.axis_index("i")

    barrier = pltpu.get_barrier_semaphore()
    for j in range(1, N_DEV):
        pl.semaphore_signal(barrier, inc=1,
                            device_id=(lax.rem(my + j, N_DEV),),
                            device_id_type=pl.DeviceIdType.MESH)

    acc = jnp.zeros((SLOCAL, D_OUT), jnp.float32)
    for k in range(E_LOCAL):
        acc += jnp.dot(xgm_ref[k], w_ref[k],
                       preferred_element_type=jnp.float32)
    ag_ref[pl.ds(my * SLOCAL, SLOCAL), :] = acc.astype(jnp.bfloat16)

    pl.semaphore_wait(barrier, N_DEV - 1)

    sends = []
    for j in range(1, N_DEV):
        rdma = pltpu.make_async_remote_copy(
            src_ref=ag_ref.at[pl.ds(my * SLOCAL, SLOCAL)],
            dst_ref=ag_ref.at[pl.ds(my * SLOCAL, SLOCAL)],
            send_sem=ssem.at[j - 1],
            recv_sem=rsem.at[j - 1],
            device_id=(lax.rem(my + j, N_DEV),),
            device_id_type=pl.DeviceIdType.MESH,
        )
        rdma.start()
        sends.append(rdma)

    for j in range(1, N_DEV):
        peer = lax.rem(my - j + N_DEV, N_DEV)
        recv = pltpu.make_async_remote_copy(
            src_ref=ag_ref.at[pl.ds(my * SLOCAL, SLOCAL)],
            dst_ref=ag_ref.at[pl.ds(peer * SLOCAL, SLOCAL)],
            send_sem=ssem.at[j - 1],
            recv_sem=rsem.at[j - 1],
            device_id=(peer,),
            device_id_type=pl.DeviceIdType.MESH,
        )
        recv.wait_recv()

    out_ref[...] = jnp.dot(p_ref[...], ag_ref[...],
                           preferred_element_type=jnp.float32)

    for rdma in sends:
        rdma.wait_send()


def kernel(x, router_W, route_idx, expert_W):
    del router_W
    my = lax.axis_index("i")
    route = route_idx[:, 0]

    sel = route[:, None] == jnp.arange(N_EXP, dtype=route.dtype)[None, :]
    occ = jnp.take_along_axis(
        jnp.cumsum(sel.astype(jnp.int32), axis=0), route[:, None], axis=1
    )[:, 0]
    kept = occ <= CAP
    gslot = route * CAP + occ - 1
    tok = jnp.arange(N_TOK, dtype=jnp.int32)

    slot_token = jnp.full((STOT,), -1, jnp.int32).at[
        jnp.where(kept, gslot, STOT)
    ].set(tok, mode="drop")

    stl = lax.dynamic_slice(slot_token, (my * SLOCAL,), (SLOCAL,))
    xg = jnp.where((stl >= 0)[:, None],
                   x[jnp.clip(stl, 0, N_TOK - 1)], 0.0)
    srow = jnp.arange(SLOCAL)
    gmask = ((srow[None, :] >= CAP * jnp.arange(E_LOCAL)[:, None])
             & (srow[None, :] < CAP * (jnp.arange(E_LOCAL)[:, None] + 1)))
    xgm = (xg[None] * gmask[:, :, None]).astype(jnp.bfloat16)

    p = (slot_token[None, :] == tok[:, None]).astype(jnp.bfloat16)

    wb = expert_W.astype(jnp.bfloat16)

    return pl.pallas_call(
        _body,
        out_shape=jax.ShapeDtypeStruct((N_TOK, D_OUT), jnp.float32),
        in_specs=[pl.BlockSpec(memory_space=pltpu.VMEM)] * 3,
        out_specs=pl.BlockSpec(memory_space=pltpu.VMEM),
        scratch_shapes=[
            pltpu.VMEM((STOT, D_OUT), jnp.bfloat16),
            pltpu.SemaphoreType.DMA((N_DEV - 1,)),
            pltpu.SemaphoreType.DMA((N_DEV - 1,)),
        ],
        compiler_params=pltpu.CompilerParams(collective_id=0),
    )(xgm, wb, p)


# device time: 32267 ns/iter; 3.6185x vs baseline; 1.8908x over previous
import jax
import jax.numpy as jnp
from jax import lax
from jax.experimental import pallas as pl
from jax.experimental.pallas import tpu as pltpu

N_DEV = 16
N_TOK = 1024
D_IN = 512
D_OUT = 1024
N_EXP = 64
E_LOCAL = 4
CAP = 12
SLOCAL = E_LOCAL * CAP
STOT = N_DEV * SLOCAL


def _body(x_ref, route_ref, w_ref, out_ref, ag_ref, ssem, rsem):
    f32 = jnp.float32
    bf16 = jnp.bfloat16
    my = lax.axis_index("i")

    barrier = pltpu.get_barrier_semaphore()
    for j in range(1, N_DEV):
        pl.semaphore_signal(barrier, inc=1,
                            device_id=(lax.rem(my + j, N_DEV),),
                            device_id_type=pl.DeviceIdType.MESH)

    i32 = jnp.int32
    route = route_ref[...]
    e_iota = lax.broadcasted_iota(i32, (N_TOK, N_EXP), 1)
    sel = (route == e_iota).astype(bf16)
    r_iota = lax.broadcasted_iota(i32, (N_TOK, N_TOK), 0)
    c_iota = lax.broadcasted_iota(i32, (N_TOK, N_TOK), 1)
    tril = (r_iota >= c_iota).astype(bf16)
    occ64 = jnp.dot(tril, sel, preferred_element_type=f32)
    occ = jnp.sum(occ64 * sel.astype(f32), axis=1,
                  keepdims=True).astype(i32)
    kept = occ <= CAP
    gslot = route * CAP + occ - 1

    t_iota = lax.broadcasted_iota(i32, (N_TOK, SLOCAL), 1)
    pmy = ((gslot - my * SLOCAL == t_iota) & kept).astype(bf16)
    xb = x_ref[...].astype(bf16)
    xg = lax.dot_general(pmy, xb, (((0,), (0,)), ((), ())),
                         preferred_element_type=f32).astype(bf16)

    g_iota = lax.broadcasted_iota(jnp.int32, (SLOCAL, 1), 0)
    acc = jnp.zeros((SLOCAL, D_OUT), f32)
    for k in range(E_LOCAL):
        gmask = ((g_iota >= k * CAP) & (g_iota < (k + 1) * CAP))
        acc += jnp.dot(xg * gmask.astype(bf16), w_ref[k].astype(bf16),
                       preferred_element_type=f32)
    ag_ref[pl.ds(my * SLOCAL, SLOCAL), :] = acc.astype(bf16)

    pl.semaphore_wait(barrier, N_DEV - 1)

    sends = []
    for j in range(1, N_DEV):
        rdma = pltpu.make_async_remote_copy(
            src_ref=ag_ref.at[pl.ds(my * SLOCAL, SLOCAL)],
            dst_ref=ag_ref.at[pl.ds(my * SLOCAL, SLOCAL)],
            send_sem=ssem.at[j - 1],
            recv_sem=rsem.at[j - 1],
            device_id=(lax.rem(my + j, N_DEV),),
            device_id_type=pl.DeviceIdType.MESH,
        )
        rdma.start()
        sends.append(rdma)

    s_iota = lax.broadcasted_iota(jnp.int32, (N_TOK, STOT), 1)
    p = ((gslot == s_iota) & kept).astype(bf16)

    for j in range(1, N_DEV):
        peer = lax.rem(my - j + N_DEV, N_DEV)
        recv = pltpu.make_async_remote_copy(
            src_ref=ag_ref.at[pl.ds(my * SLOCAL, SLOCAL)],
            dst_ref=ag_ref.at[pl.ds(peer * SLOCAL, SLOCAL)],
            send_sem=ssem.at[j - 1],
            recv_sem=rsem.at[j - 1],
            device_id=(peer,),
            device_id_type=pl.DeviceIdType.MESH,
        )
        recv.wait_recv()

    out_ref[...] = jnp.dot(p, ag_ref[...], preferred_element_type=f32)

    for rdma in sends:
        rdma.wait_send()


def kernel(x, router_W, route_idx, expert_W):
    del router_W
    return pl.pallas_call(
        _body,
        out_shape=jax.ShapeDtypeStruct((N_TOK, D_OUT), jnp.float32),
        in_specs=[pl.BlockSpec(memory_space=pltpu.VMEM)] * 3,
        out_specs=pl.BlockSpec(memory_space=pltpu.VMEM),
        scratch_shapes=[
            pltpu.VMEM((STOT, D_OUT), jnp.bfloat16),
            pltpu.SemaphoreType.DMA((N_DEV - 1,)),
            pltpu.SemaphoreType.DMA((N_DEV - 1,)),
        ],
        compiler_params=pltpu.CompilerParams(collective_id=0),
    )(x, route_idx, expert_W)
